# baseline (speedup 1.0000x reference)
import functools

import jax
import jax.numpy as jnp
from jax import lax
from jax.experimental import pallas as pl
from jax.experimental.pallas import tpu as pltpu
from jax.experimental.pallas import tpu_sc as plsc

_B = 16
_P = 4096
_D = 192
_NC = 2
_NS = 16
_GROUPS = _D // 8


def _make_sc_kernel():
    mesh = plsc.VectorSubcoreMesh(core_axis_name="c", subcore_axis_name="s")

    @functools.partial(
        pl.kernel,
        mesh=mesh,
        compiler_params=pltpu.CompilerParams(use_tc_tiling_on_sc=True),
        out_type=jax.ShapeDtypeStruct((_B, _D, _P), jnp.float32),
        scratch_types=[
            pltpu.VMEM((8, _P), jnp.float32),
            pltpu.SemaphoreType.DMA,
        ],
    )
    def k(table_hbm, out_hbm, buf, sem):
        wid = lax.axis_index("s") * _NC + lax.axis_index("c")

        @pl.when(wid < _GROUPS)
        def _():
            base = wid * 8
            pltpu.sync_copy(table_hbm.at[pl.ds(base, 8)], buf)
            handles = [
                pltpu.async_copy(buf, out_hbm.at[b, pl.ds(base, 8)], sem)
                for b in range(_B)
            ]
            for h in handles:
                h.wait()

    return k


_sc_broadcast = _make_sc_kernel()


def kernel(x, pos_table):
    del x
    out_t = _sc_broadcast(pos_table.T)
    return jnp.transpose(out_t, (0, 2, 1))


# speedup vs baseline: 2.2821x; 2.2821x over previous
import jax
import jax.numpy as jnp
from jax.experimental import pallas as pl
from jax.experimental.pallas import tpu as pltpu

_NCHUNK = 4


def _body(t_hbm, o_hbm, buf, sem_in, sem_out):
    B, D, P = o_hbm.shape
    rows = D // _NCHUNK
    loads = [
        pltpu.make_async_copy(
            t_hbm.at[pl.ds(i * rows, rows)],
            buf.at[pl.ds(i * rows, rows)],
            sem_in,
        )
        for i in range(_NCHUNK)
    ]
    for ld in loads:
        ld.start()
    writes = []
    for i in range(_NCHUNK):
        loads[i].wait()
        for b in range(B):
            c = pltpu.make_async_copy(
                buf.at[pl.ds(i * rows, rows)],
                o_hbm.at[b, pl.ds(i * rows, rows)],
                sem_out.at[b],
            )
            c.start()
            writes.append(c)
    for c in writes:
        c.wait()


def kernel(x, pos_table):
    B = x.shape[0]
    P, D = pos_table.shape
    table_t = pos_table.T
    out_t = pl.pallas_call(
        _body,
        in_specs=[pl.BlockSpec(memory_space=pl.ANY)],
        out_specs=pl.BlockSpec(memory_space=pl.ANY),
        out_shape=jax.ShapeDtypeStruct((B, D, P), jnp.float32),
        scratch_shapes=[
            pltpu.VMEM((D, P), jnp.float32),
            pltpu.SemaphoreType.DMA,
            pltpu.SemaphoreType.DMA((B,)),
        ],
    )(table_t)
    return jnp.transpose(out_t, (0, 2, 1))
